# Initial kernel scaffold; baseline (speedup 1.0000x reference)
import functools

import jax
import jax.numpy as jnp
from jax import lax
from jax.experimental import pallas as pl
from jax.experimental.pallas import tpu as pltpu
from jax.experimental.pallas import tpu_sc as plsc

NUM_E = 8192
D = 32
N_ROWS = 9216
ROW_BLK = 1152
N_BLKS = N_ROWS // ROW_BLK
E_CHUNK = 1024
N_CHUNKS = NUM_E // E_CHUNK

NC, NS = 2, 16
NW = NC * NS
BPW = N_ROWS // NW


def _argmin_body(xt_ref, e_ref, idx_ref, dsum_ref):
    b = pl.program_id(0)
    xt = xt_ref[...]
    a = jnp.sum(xt * xt, axis=0, keepdims=True)
    run_min = jnp.full((1, ROW_BLK), jnp.inf, jnp.float32)
    run_idx = jnp.zeros((1, ROW_BLK), jnp.int32)
    for c in range(N_CHUNKS):
        ec = e_ref[pl.ds(c * E_CHUNK, E_CHUNK), :]
        bn = jnp.sum(ec * ec, axis=1, keepdims=True)
        m = lax.dot_general(ec, xt, (((1,), (0,)), ((), ())),
                            preferred_element_type=jnp.float32)
        s = (a + bn) - 2.0 * m
        minv = jnp.min(s, axis=0, keepdims=True)
        iota = lax.broadcasted_iota(jnp.int32, (E_CHUNK, ROW_BLK), 0) + c * E_CHUNK
        idxc = jnp.min(jnp.where(s == minv, iota, jnp.int32(2 ** 30)),
                       axis=0, keepdims=True)
        upd = minv < run_min
        run_min = jnp.where(upd, minv, run_min)
        run_idx = jnp.where(upd, idxc, run_idx)
    idx_ref[...] = run_idx.reshape(1, 1, ROW_BLK)

    @pl.when(b == 0)
    def _():
        dsum_ref[...] = jnp.zeros((1, 1), jnp.float32)

    dsum_ref[...] += jnp.sum(run_min).reshape(1, 1)


_argmin_call = pl.pallas_call(
    _argmin_body,
    grid=(N_BLKS,),
    in_specs=[
        pl.BlockSpec((D, ROW_BLK), lambda b: (0, b)),
        pl.BlockSpec((NUM_E, D), lambda b: (0, 0)),
    ],
    out_specs=[
        pl.BlockSpec((1, 1, ROW_BLK), lambda b: (b, 0, 0)),
        pl.BlockSpec((1, 1), lambda b: (0, 0)),
    ],
    out_shape=[
        jax.ShapeDtypeStruct((N_BLKS, 1, ROW_BLK), jnp.int32),
        jax.ShapeDtypeStruct((1, 1), jnp.float32),
    ],
)


def _sc_gather_body(idx_hbm, x_hbm, tab_hbm, out_hbm, idx_v, q_v, x_v, sem):
    wid = lax.axis_index("s") * NC + lax.axis_index("c")
    base = wid * BPW
    pltpu.sync_copy(idx_hbm.at[pl.ds(base, BPW)], idx_v)
    gather = pltpu.async_copy(tab_hbm.at[idx_v], q_v, sem)
    pltpu.sync_copy(x_hbm.at[pl.ds(base, BPW)], x_v)
    gather.wait()

    def body(i, carry):
        for h in range(D // 16):
            xv = x_v[i, pl.ds(h * 16, 16)]
            qv = q_v[i, pl.ds(h * 16, 16)]
            q_v[i, pl.ds(h * 16, 16)] = xv + (qv - xv)
        return carry

    lax.fori_loop(0, BPW, body, 0)
    pltpu.sync_copy(q_v, out_hbm.at[pl.ds(base, BPW)])


_sc_gather_call = pl.kernel(
    _sc_gather_body,
    out_type=jax.ShapeDtypeStruct((N_ROWS, D), jnp.float32),
    mesh=plsc.VectorSubcoreMesh(core_axis_name="c", subcore_axis_name="s"),
    scratch_types=[
        pltpu.VMEM((BPW,), jnp.int32),
        pltpu.VMEM((BPW, D), jnp.float32),
        pltpu.VMEM((BPW, D), jnp.float32),
        pltpu.SemaphoreType.DMA,
    ],
)


def kernel(input, embed_weight):
    flat = input.reshape(N_ROWS, D)
    idx_blk, dsum = _argmin_call(flat.T, embed_weight)
    idx = idx_blk.reshape(N_ROWS)
    q = _sc_gather_call(idx, flat, embed_weight)
    loss = 0.25 * (dsum[0, 0] / jnp.float32(N_ROWS * D))
    return q.reshape(input.shape), loss


# speedup vs baseline: 3.7283x; 3.7283x over previous
import functools

import jax
import jax.numpy as jnp
from jax import lax
from jax.experimental import pallas as pl
from jax.experimental.pallas import tpu as pltpu
from jax.experimental.pallas import tpu_sc as plsc

NUM_E = 8192
D = 32
N_ROWS = 9216
ROW_BLK = 1152
N_BLKS = N_ROWS // ROW_BLK
E_CHUNK = 1024
N_CHUNKS = NUM_E // E_CHUNK

NC, NS = 2, 16
NW = NC * NS
BPW = N_ROWS // NW


def _argmin_body(xt_ref, e_ref, idx_ref, dsum_ref):
    b = pl.program_id(0)
    xt = xt_ref[...]
    a = jnp.sum(xt * xt, axis=0, keepdims=True)
    run_min = jnp.full((1, ROW_BLK), jnp.inf, jnp.float32)
    run_idx = jnp.zeros((1, ROW_BLK), jnp.int32)
    for c in range(N_CHUNKS):
        ec = e_ref[pl.ds(c * E_CHUNK, E_CHUNK), :]
        bn = jnp.sum(ec * ec, axis=1, keepdims=True)
        m = lax.dot_general(ec, xt, (((1,), (0,)), ((), ())),
                            preferred_element_type=jnp.float32)
        s = (a + bn) - 2.0 * m
        minv = jnp.min(s, axis=0, keepdims=True)
        iota = lax.broadcasted_iota(jnp.int32, (E_CHUNK, ROW_BLK), 0) + c * E_CHUNK
        idxc = jnp.min(jnp.where(s == minv, iota, jnp.int32(2 ** 30)),
                       axis=0, keepdims=True)
        upd = minv < run_min
        run_min = jnp.where(upd, minv, run_min)
        run_idx = jnp.where(upd, idxc, run_idx)
    idx_ref[...] = run_idx.reshape(1, 1, ROW_BLK)

    @pl.when(b == 0)
    def _():
        dsum_ref[...] = jnp.zeros((1, 1), jnp.float32)

    dsum_ref[...] += jnp.sum(run_min).reshape(1, 1)


_argmin_call = pl.pallas_call(
    _argmin_body,
    grid=(N_BLKS,),
    in_specs=[
        pl.BlockSpec((D, ROW_BLK), lambda b: (0, b)),
        pl.BlockSpec((NUM_E, D), lambda b: (0, 0)),
    ],
    out_specs=[
        pl.BlockSpec((1, 1, ROW_BLK), lambda b: (b, 0, 0)),
        pl.BlockSpec((1, 1), lambda b: (0, 0)),
    ],
    out_shape=[
        jax.ShapeDtypeStruct((N_BLKS, 1, ROW_BLK), jnp.int32),
        jax.ShapeDtypeStruct((1, 1), jnp.float32),
    ],
)


def _sc_gather_body(idx_hbm, x_hbm, tab_hbm, out_hbm, idx_v, q_v, x_v, sem):
    wid = lax.axis_index("s") * NC + lax.axis_index("c")
    base = wid * BPW
    pltpu.sync_copy(idx_hbm.at[pl.ds(base, BPW)], idx_v)
    gather = pltpu.async_copy(tab_hbm.at[idx_v], q_v, sem)
    pltpu.sync_copy(x_hbm.at[pl.ds(base, BPW)], x_v)
    gather.wait()

    def body(i, carry):
        for h in range(D // 16):
            xv = x_v[i, pl.ds(h * 16, 16)]
            qv = q_v[i, pl.ds(h * 16, 16)]
            q_v[i, pl.ds(h * 16, 16)] = xv + (qv - xv)
        return carry

    lax.fori_loop(0, BPW, body, 0)
    pltpu.sync_copy(q_v, out_hbm.at[pl.ds(base, BPW)])


@functools.cache
def _sc_gather_call():
    return pl.kernel(
        _sc_gather_body,
        out_type=jax.ShapeDtypeStruct((N_ROWS, D), jnp.float32),
        mesh=plsc.VectorSubcoreMesh(
            core_axis_name="c", subcore_axis_name="s",
            num_cores=NC, num_subcores=NS),
        scratch_types=[
            pltpu.VMEM((BPW,), jnp.int32),
            pltpu.VMEM((BPW, D), jnp.float32),
            pltpu.VMEM((BPW, D), jnp.float32),
            pltpu.SemaphoreType.DMA,
        ],
        compiler_params=pltpu.CompilerParams(use_tc_tiling_on_sc=False),
    )


def kernel(input, embed_weight):
    flat = input.reshape(N_ROWS, D)
    idx_blk, dsum = _argmin_call(flat.T, embed_weight)
    idx = idx_blk.reshape(N_ROWS)
    q = _sc_gather_call()(idx, flat, embed_weight)
    loss = 0.25 * (dsum[0, 0] / jnp.float32(N_ROWS * D))
    return q.reshape(input.shape), loss


# speedup vs baseline: 3.8897x; 1.0433x over previous
import functools

import jax
import jax.numpy as jnp
from jax import lax
from jax.experimental import pallas as pl
from jax.experimental.pallas import tpu as pltpu
from jax.experimental.pallas import tpu_sc as plsc

NUM_E = 8192
D = 32
N_ROWS = 9216
ROW_BLK = 1152
N_BLKS = N_ROWS // ROW_BLK
E_CHUNK = 1024
N_CHUNKS = NUM_E // E_CHUNK

NC, NS = 2, 16
NW = NC * NS
BPW = N_ROWS // NW


def _argmin_body(xt_ref, e_ref, idx_ref, dsum_ref):
    b = pl.program_id(0)
    xt = xt_ref[...]
    a = jnp.sum(xt * xt, axis=0, keepdims=True)
    run_min = jnp.full((1, ROW_BLK), jnp.inf, jnp.float32)
    run_idx = jnp.zeros((1, ROW_BLK), jnp.int32)
    for c in range(N_CHUNKS):
        ec = e_ref[pl.ds(c * E_CHUNK, E_CHUNK), :]
        bn = jnp.sum(ec * ec, axis=1, keepdims=True)
        m2 = lax.dot_general(ec + ec, xt, (((1,), (0,)), ((), ())),
                             preferred_element_type=jnp.float32)
        s = (a + bn) - m2
        minv = jnp.min(s, axis=0, keepdims=True)
        iota = lax.broadcasted_iota(jnp.int32, (E_CHUNK, ROW_BLK), 0) + c * E_CHUNK
        idxc = jnp.min(jnp.where(s == minv, iota, jnp.int32(2 ** 30)),
                       axis=0, keepdims=True)
        upd = minv < run_min
        run_min = jnp.where(upd, minv, run_min)
        run_idx = jnp.where(upd, idxc, run_idx)
    idx_ref[...] = run_idx.reshape(1, 1, ROW_BLK)

    @pl.when(b == 0)
    def _():
        dsum_ref[...] = jnp.zeros((1, 1), jnp.float32)

    dsum_ref[...] += jnp.sum(run_min).reshape(1, 1)


_argmin_call = pl.pallas_call(
    _argmin_body,
    grid=(N_BLKS,),
    in_specs=[
        pl.BlockSpec((D, ROW_BLK), lambda b: (0, b)),
        pl.BlockSpec((NUM_E, D), lambda b: (0, 0)),
    ],
    out_specs=[
        pl.BlockSpec((1, 1, ROW_BLK), lambda b: (b, 0, 0)),
        pl.BlockSpec((1, 1), lambda b: (0, 0)),
    ],
    out_shape=[
        jax.ShapeDtypeStruct((N_BLKS, 1, ROW_BLK), jnp.int32),
        jax.ShapeDtypeStruct((1, 1), jnp.float32),
    ],
)


def _sc_gather_body(idx_hbm, x_hbm, tab_hbm, out_hbm, idx_v, q_v, x_v, sem):
    wid = lax.axis_index("s") * NC + lax.axis_index("c")
    base = wid * BPW
    pltpu.sync_copy(idx_hbm.at[pl.ds(base, BPW)], idx_v)
    gather = pltpu.async_copy(tab_hbm.at[idx_v], q_v, sem)
    pltpu.sync_copy(x_hbm.at[pl.ds(base, BPW)], x_v)
    gather.wait()

    def body(i, carry):
        for h in range(D // 16):
            xv = x_v[i, pl.ds(h * 16, 16)]
            qv = q_v[i, pl.ds(h * 16, 16)]
            q_v[i, pl.ds(h * 16, 16)] = xv + (qv - xv)
        return carry

    lax.fori_loop(0, BPW, body, 0)
    pltpu.sync_copy(q_v, out_hbm.at[pl.ds(base, BPW)])


@functools.cache
def _sc_gather_call():
    return pl.kernel(
        _sc_gather_body,
        out_type=jax.ShapeDtypeStruct((N_ROWS, D), jnp.float32),
        mesh=plsc.VectorSubcoreMesh(
            core_axis_name="c", subcore_axis_name="s",
            num_cores=NC, num_subcores=NS),
        scratch_types=[
            pltpu.VMEM((BPW,), jnp.int32),
            pltpu.VMEM((BPW, D), jnp.float32),
            pltpu.VMEM((BPW, D), jnp.float32),
            pltpu.SemaphoreType.DMA,
        ],
        compiler_params=pltpu.CompilerParams(use_tc_tiling_on_sc=False),
    )


def kernel(input, embed_weight):
    flat = input.reshape(N_ROWS, D)
    idx_blk, dsum = _argmin_call(flat.T, embed_weight)
    idx = idx_blk.reshape(N_ROWS)
    q = _sc_gather_call()(idx, flat, embed_weight)
    loss = 0.25 * (dsum[0, 0] / jnp.float32(N_ROWS * D))
    return q.reshape(input.shape), loss


# speedup vs baseline: 3.9180x; 1.0073x over previous
import functools

import jax
import jax.numpy as jnp
from jax import lax
from jax.experimental import pallas as pl
from jax.experimental.pallas import tpu as pltpu
from jax.experimental.pallas import tpu_sc as plsc

NUM_E = 8192
D = 32
N_ROWS = 9216
ROW_BLK = 1152
N_BLKS = N_ROWS // ROW_BLK
E_CHUNK = 1024
N_CHUNKS = NUM_E // E_CHUNK

NC, NS = 2, 16
NW = NC * NS
BPW = N_ROWS // NW


def _argmin_body(xt_ref, e_ref, idx_ref, dsum_ref):
    b = pl.program_id(0)
    xt = xt_ref[...]
    a = jnp.sum(xt * xt, axis=0, keepdims=True)
    run_min = jnp.full((1, ROW_BLK), jnp.inf, jnp.float32)
    run_idx = jnp.zeros((1, ROW_BLK), jnp.int32)
    iota = lax.broadcasted_iota(jnp.int32, (E_CHUNK, ROW_BLK), 0)
    for c in range(N_CHUNKS):
        ec = e_ref[pl.ds(c * E_CHUNK, E_CHUNK), :]
        bn = jnp.sum(ec * ec, axis=1, keepdims=True)
        m2 = lax.dot_general(ec + ec, xt, (((1,), (0,)), ((), ())),
                             preferred_element_type=jnp.float32)
        s = (a + bn) - m2
        minv = jnp.min(s, axis=0, keepdims=True)
        idxc = jnp.min(jnp.where(s == minv, iota, jnp.int32(2 ** 30)),
                       axis=0, keepdims=True) + c * E_CHUNK
        upd = minv < run_min
        run_min = jnp.where(upd, minv, run_min)
        run_idx = jnp.where(upd, idxc, run_idx)
    idx_ref[...] = run_idx.reshape(1, 1, ROW_BLK)

    @pl.when(b == 0)
    def _():
        dsum_ref[...] = jnp.zeros((1, 1), jnp.float32)

    dsum_ref[...] += jnp.sum(run_min).reshape(1, 1)


_argmin_call = pl.pallas_call(
    _argmin_body,
    grid=(N_BLKS,),
    in_specs=[
        pl.BlockSpec((D, ROW_BLK), lambda b: (0, b)),
        pl.BlockSpec((NUM_E, D), lambda b: (0, 0)),
    ],
    out_specs=[
        pl.BlockSpec((1, 1, ROW_BLK), lambda b: (b, 0, 0)),
        pl.BlockSpec((1, 1), lambda b: (0, 0)),
    ],
    out_shape=[
        jax.ShapeDtypeStruct((N_BLKS, 1, ROW_BLK), jnp.int32),
        jax.ShapeDtypeStruct((1, 1), jnp.float32),
    ],
)


def _sc_gather_body(idx_hbm, x_hbm, tab_hbm, out_hbm, idx_v, q_v, x_v, sem):
    wid = lax.axis_index("s") * NC + lax.axis_index("c")
    base = wid * BPW
    pltpu.sync_copy(idx_hbm.at[pl.ds(base, BPW)], idx_v)
    gather = pltpu.async_copy(tab_hbm.at[idx_v], q_v, sem)
    pltpu.sync_copy(x_hbm.at[pl.ds(base, BPW)], x_v)
    gather.wait()

    def body(i, carry):
        for h in range(D // 16):
            xv = x_v[i, pl.ds(h * 16, 16)]
            qv = q_v[i, pl.ds(h * 16, 16)]
            q_v[i, pl.ds(h * 16, 16)] = xv + (qv - xv)
        return carry

    lax.fori_loop(0, BPW, body, 0)
    pltpu.sync_copy(q_v, out_hbm.at[pl.ds(base, BPW)])


@functools.cache
def _sc_gather_call():
    return pl.kernel(
        _sc_gather_body,
        out_type=jax.ShapeDtypeStruct((N_ROWS, D), jnp.float32),
        mesh=plsc.VectorSubcoreMesh(
            core_axis_name="c", subcore_axis_name="s",
            num_cores=NC, num_subcores=NS),
        scratch_types=[
            pltpu.VMEM((BPW,), jnp.int32),
            pltpu.VMEM((BPW, D), jnp.float32),
            pltpu.VMEM((BPW, D), jnp.float32),
            pltpu.SemaphoreType.DMA,
        ],
        compiler_params=pltpu.CompilerParams(use_tc_tiling_on_sc=False),
    )


def kernel(input, embed_weight):
    flat = input.reshape(N_ROWS, D)
    idx_blk, dsum = _argmin_call(flat.T, embed_weight)
    idx = idx_blk.reshape(N_ROWS)
    q = _sc_gather_call()(idx, flat, embed_weight)
    loss = 0.25 * (dsum[0, 0] / jnp.float32(N_ROWS * D))
    return q.reshape(input.shape), loss


# speedup vs baseline: 4.1539x; 1.0602x over previous
import functools

import jax
import jax.numpy as jnp
from jax import lax
from jax.experimental import pallas as pl
from jax.experimental.pallas import tpu as pltpu
from jax.experimental.pallas import tpu_sc as plsc

NUM_E = 8192
D = 32
N_ROWS = 9216
ROW_BLK = 1152
N_BLKS = N_ROWS // ROW_BLK
E_CHUNK = 1024
N_CHUNKS = NUM_E // E_CHUNK

NC, NS = 2, 16
NW = NC * NS
BPW = N_ROWS // NW


def _argmin_body(xt_ref, e_ref, idx_ref, dsum_ref):
    b = pl.program_id(0)
    xt = xt_ref[...]
    a = jnp.sum(xt * xt, axis=0, keepdims=True)
    run_min = jnp.full((1, ROW_BLK), jnp.inf, jnp.float32)
    run_idx = jnp.zeros((1, ROW_BLK), jnp.float32)
    iota = lax.broadcasted_iota(jnp.int32, (E_CHUNK, ROW_BLK), 0).astype(jnp.float32)
    for c in range(N_CHUNKS):
        ec = e_ref[pl.ds(c * E_CHUNK, E_CHUNK), :]
        bn = jnp.sum(ec * ec, axis=1, keepdims=True)
        m2 = lax.dot_general(ec + ec, xt, (((1,), (0,)), ((), ())),
                             preferred_element_type=jnp.float32)
        s = (a + bn) - m2
        minv = jnp.min(s, axis=0, keepdims=True)
        idxc = jnp.min(jnp.where(s == minv, iota, jnp.float32(2.0 ** 24)),
                       axis=0, keepdims=True) + jnp.float32(c * E_CHUNK)
        upd = minv < run_min
        run_min = jnp.where(upd, minv, run_min)
        run_idx = jnp.where(upd, idxc, run_idx)
    idx_ref[...] = run_idx.astype(jnp.int32).reshape(1, 1, ROW_BLK)

    @pl.when(b == 0)
    def _():
        dsum_ref[...] = jnp.zeros((1, 1), jnp.float32)

    dsum_ref[...] += jnp.sum(run_min).reshape(1, 1)


_argmin_call = pl.pallas_call(
    _argmin_body,
    grid=(N_BLKS,),
    in_specs=[
        pl.BlockSpec((D, ROW_BLK), lambda b: (0, b)),
        pl.BlockSpec((NUM_E, D), lambda b: (0, 0)),
    ],
    out_specs=[
        pl.BlockSpec((1, 1, ROW_BLK), lambda b: (b, 0, 0)),
        pl.BlockSpec((1, 1), lambda b: (0, 0)),
    ],
    out_shape=[
        jax.ShapeDtypeStruct((N_BLKS, 1, ROW_BLK), jnp.int32),
        jax.ShapeDtypeStruct((1, 1), jnp.float32),
    ],
)


def _sc_gather_body(idx_hbm, x_hbm, tab_hbm, out_hbm, idx_v, q_v, x_v, sem):
    wid = lax.axis_index("s") * NC + lax.axis_index("c")
    base = wid * BPW
    pltpu.sync_copy(idx_hbm.at[pl.ds(base, BPW)], idx_v)
    gather = pltpu.async_copy(tab_hbm.at[idx_v], q_v, sem)
    pltpu.sync_copy(x_hbm.at[pl.ds(base, BPW)], x_v)
    gather.wait()

    def body(i, carry):
        for h in range(D // 16):
            xv = x_v[i, pl.ds(h * 16, 16)]
            qv = q_v[i, pl.ds(h * 16, 16)]
            q_v[i, pl.ds(h * 16, 16)] = xv + (qv - xv)
        return carry

    lax.fori_loop(0, BPW, body, 0)
    pltpu.sync_copy(q_v, out_hbm.at[pl.ds(base, BPW)])


@functools.cache
def _sc_gather_call():
    return pl.kernel(
        _sc_gather_body,
        out_type=jax.ShapeDtypeStruct((N_ROWS, D), jnp.float32),
        mesh=plsc.VectorSubcoreMesh(
            core_axis_name="c", subcore_axis_name="s",
            num_cores=NC, num_subcores=NS),
        scratch_types=[
            pltpu.VMEM((BPW,), jnp.int32),
            pltpu.VMEM((BPW, D), jnp.float32),
            pltpu.VMEM((BPW, D), jnp.float32),
            pltpu.SemaphoreType.DMA,
        ],
        compiler_params=pltpu.CompilerParams(use_tc_tiling_on_sc=False),
    )


def kernel(input, embed_weight):
    flat = input.reshape(N_ROWS, D)
    idx_blk, dsum = _argmin_call(flat.T, embed_weight)
    idx = idx_blk.reshape(N_ROWS)
    q = _sc_gather_call()(idx, flat, embed_weight)
    loss = 0.25 * (dsum[0, 0] / jnp.float32(N_ROWS * D))
    return q.reshape(input.shape), loss
